# Initial kernel scaffold; baseline (speedup 1.0000x reference)
import jax
import jax.numpy as jnp
from jax.experimental import pallas as pl


def kernel(q, k):
    raise NotImplementedError("write your pallas kernel here")


# speedup vs baseline: 24.2394x; 24.2394x over previous
import jax
import jax.numpy as jnp
from jax.experimental import pallas as pl

B, L, S, H, G, D = 2, 4096, 128, 4, 4, 128
TOPK = 16
BLOCK = 32
WINDOW = 32
SM_SCALE = 1.0 / (D ** 0.5)

TL = 512


def _topk_kernel(q_ref, k_ref, out_ref, idx_ref):
    lt = pl.program_id(1)

    l_col = lt * TL + jax.lax.broadcasted_iota(jnp.int32, (TL, 1), 0)
    lim = ((l_col + 1) // BLOCK) - 1
    s_iota = jax.lax.broadcasted_iota(jnp.int32, (TL, S), 1)
    smask = s_iota >= lim
    neg_inf = jnp.float32(-jnp.inf)

    for h in range(H):
        qall = q_ref[0, :, h, :, :].reshape(TL * G, D)
        kb = k_ref[0, h, :, :]

        scores = jax.lax.dot_general(
            qall, kb, (((1,), (1,)), ((), ())),
            preferred_element_type=jnp.float32)

        s3 = scores.reshape(TL, G, S)
        scores_g = [s3[:, g, :] for g in range(G)]

        pooled = jnp.maximum(jnp.maximum(scores_g[0], scores_g[1]),
                             jnp.maximum(scores_g[2], scores_g[3]))
        pooled = jnp.where(smask, neg_inf, pooled)

        idx_cols = []
        out_cols = [[] for _ in range(G)]
        for _ in range(TOPK):
            m = jnp.max(pooled, axis=-1, keepdims=True)
            i = jnp.min(jnp.where(pooled == m, s_iota, S), axis=-1,
                        keepdims=True)
            onehot = s_iota == i
            invalid = jnp.isneginf(m)
            idx_cols.append(jnp.where(invalid, -1, i))
            for g in range(G):
                v = jnp.sum(jnp.where(onehot, scores_g[g], 0.0), axis=-1,
                            keepdims=True)
                out_cols[g].append(jnp.where(invalid, 0.0, v * SM_SCALE))
            pooled = jnp.where(onehot, neg_inf, pooled)

        idx_ref[0, :, h, :] = jnp.concatenate(idx_cols, axis=-1)
        for g in range(G):
            out_ref[0, :, h, g, :] = jnp.concatenate(out_cols[g], axis=-1)


@jax.jit
def kernel(q, k):
    kp = jnp.transpose(k, (0, 2, 1, 3))
    grid = (B, L // TL)
    out_scores, idx = pl.pallas_call(
        _topk_kernel,
        grid=grid,
        in_specs=[
            pl.BlockSpec((1, TL, H, G, D), lambda b, lt: (b, lt, 0, 0, 0)),
            pl.BlockSpec((1, H, S, D), lambda b, lt: (b, 0, 0, 0)),
        ],
        out_specs=[
            pl.BlockSpec((1, TL, H, G, TOPK), lambda b, lt: (b, lt, 0, 0, 0)),
            pl.BlockSpec((1, TL, H, TOPK), lambda b, lt: (b, lt, 0, 0)),
        ],
        out_shape=[
            jax.ShapeDtypeStruct((B, L, H, G, TOPK), jnp.float32),
            jax.ShapeDtypeStruct((B, L, H, TOPK), jnp.int32),
        ],
    )(q, kp)
    return out_scores, idx


# speedup vs baseline: 94.0021x; 3.8781x over previous
import jax
import jax.numpy as jnp
from jax.experimental import pallas as pl

B, L, S, H, G, D = 2, 4096, 128, 4, 4, 128
TOPK = 16
BLOCK = 32
WINDOW = 32
SM_SCALE = 1.0 / (D ** 0.5)

TL = 512


def _topk_kernel(q_ref, k_ref, out_ref, idx_ref):
    lt = pl.program_id(1)

    l_row = lt * TL + jax.lax.broadcasted_iota(jnp.int32, (1, TL), 1)
    lim = ((l_row + 1) // BLOCK) - 1
    s_iota = jax.lax.broadcasted_iota(jnp.int32, (S, TL), 0)
    smask = s_iota >= lim
    neg_inf = jnp.float32(-jnp.inf)

    for h in range(H):
        kb = k_ref[0, h, :, :]
        scores_g = []
        for g in range(G):
            qg = q_ref[0, :, h, g, :]
            scores_g.append(jax.lax.dot_general(
                kb, qg, (((1,), (1,)), ((), ())),
                preferred_element_type=jnp.float32))

        pooled = jnp.maximum(jnp.maximum(scores_g[0], scores_g[1]),
                             jnp.maximum(scores_g[2], scores_g[3]))
        pooled = jnp.where(smask, neg_inf, pooled)

        idx_rows = []
        out_rows = [[] for _ in range(G)]
        for _ in range(TOPK):
            m = jnp.max(pooled, axis=0, keepdims=True)
            i = jnp.min(jnp.where(pooled == m, s_iota, S), axis=0,
                        keepdims=True)
            onehot = s_iota == i
            invalid = jnp.isneginf(m)
            idx_rows.append(jnp.where(invalid, -1, i))
            for g in range(G):
                v = jnp.sum(jnp.where(onehot, scores_g[g], 0.0), axis=0,
                            keepdims=True)
                out_rows[g].append(jnp.where(invalid, 0.0, v * SM_SCALE))
            pooled = jnp.where(onehot, neg_inf, pooled)

        idx_ref[0, h, :, :] = jnp.concatenate(idx_rows, axis=0)
        for g in range(G):
            out_ref[0, h, g, :, :] = jnp.concatenate(out_rows[g], axis=0)


@jax.jit
def kernel(q, k):
    kp = jnp.transpose(k, (0, 2, 1, 3))
    grid = (B, L // TL)
    out_t, idx_t = pl.pallas_call(
        _topk_kernel,
        grid=grid,
        in_specs=[
            pl.BlockSpec((1, TL, H, G, D), lambda b, lt: (b, lt, 0, 0, 0)),
            pl.BlockSpec((1, H, S, D), lambda b, lt: (b, 0, 0, 0)),
        ],
        out_specs=[
            pl.BlockSpec((1, H, G, TOPK, TL), lambda b, lt: (b, 0, 0, 0, lt)),
            pl.BlockSpec((1, H, TOPK, TL), lambda b, lt: (b, 0, 0, lt)),
        ],
        out_shape=[
            jax.ShapeDtypeStruct((B, H, G, TOPK, L), jnp.float32),
            jax.ShapeDtypeStruct((B, H, TOPK, L), jnp.int32),
        ],
    )(q, kp)
    out_scores = jnp.transpose(out_t, (0, 4, 1, 2, 3))
    idx = jnp.transpose(idx_t, (0, 3, 1, 2))
    return out_scores, idx


# speedup vs baseline: 94.1235x; 1.0013x over previous
import jax
import jax.numpy as jnp
from jax.experimental import pallas as pl
from jax.experimental.pallas import tpu as pltpu

B, L, S, H, G, D = 2, 4096, 128, 4, 4, 128
TOPK = 16
BLOCK = 32
WINDOW = 32
SM_SCALE = 1.0 / (D ** 0.5)

TL = 512


def _topk_kernel(q_ref, k_ref, out_ref, idx_ref):
    lt = pl.program_id(1)

    l_row = lt * TL + jax.lax.broadcasted_iota(jnp.int32, (1, TL), 1)
    lim = ((l_row + 1) // BLOCK) - 1
    s_iota = jax.lax.broadcasted_iota(jnp.int32, (S, TL), 0)
    smask = s_iota >= lim
    neg_inf = jnp.float32(-jnp.inf)

    for h in range(H):
        kb = k_ref[0, h, :, :]
        scores_g = []
        for g in range(G):
            qg = q_ref[0, :, h, g, :]
            scores_g.append(jax.lax.dot_general(
                kb, qg, (((1,), (1,)), ((), ())),
                preferred_element_type=jnp.float32))

        pooled = jnp.maximum(jnp.maximum(scores_g[0], scores_g[1]),
                             jnp.maximum(scores_g[2], scores_g[3]))
        pooled = jnp.where(smask, neg_inf, pooled)

        idx_rows = []
        out_rows = [[] for _ in range(G)]
        for _ in range(TOPK):
            m = jnp.max(pooled, axis=0, keepdims=True)
            i = jnp.min(jnp.where(pooled == m, s_iota, S), axis=0,
                        keepdims=True)
            onehot = s_iota == i
            invalid = jnp.isneginf(m)
            idx_rows.append(jnp.where(invalid, -1, i))
            for g in range(G):
                v = jnp.sum(jnp.where(onehot, scores_g[g], 0.0), axis=0,
                            keepdims=True)
                out_rows[g].append(jnp.where(invalid, 0.0, v * SM_SCALE))
            pooled = jnp.where(onehot, neg_inf, pooled)

        idx_ref[0, h, :, :] = jnp.concatenate(idx_rows, axis=0)
        for g in range(G):
            out_ref[0, h, g, :, :] = jnp.concatenate(out_rows[g], axis=0)


@jax.jit
def kernel(q, k):
    kp = jnp.transpose(k, (0, 2, 1, 3))
    grid = (B, L // TL)
    out_t, idx_t = pl.pallas_call(
        _topk_kernel,
        grid=grid,
        in_specs=[
            pl.BlockSpec((1, TL, H, G, D), lambda b, lt: (b, lt, 0, 0, 0)),
            pl.BlockSpec((1, H, S, D), lambda b, lt: (b, 0, 0, 0)),
        ],
        out_specs=[
            pl.BlockSpec((1, H, G, TOPK, TL), lambda b, lt: (b, 0, 0, 0, lt)),
            pl.BlockSpec((1, H, TOPK, TL), lambda b, lt: (b, 0, 0, lt)),
        ],
        out_shape=[
            jax.ShapeDtypeStruct((B, H, G, TOPK, L), jnp.float32),
            jax.ShapeDtypeStruct((B, H, TOPK, L), jnp.int32),
        ],
        compiler_params=pltpu.CompilerParams(
            dimension_semantics=("parallel", "parallel")),
    )(q, kp)
    out_scores = jnp.transpose(out_t, (0, 4, 1, 2, 3))
    idx = jnp.transpose(idx_t, (0, 3, 1, 2))
    return out_scores, idx


# speedup vs baseline: 111.4259x; 1.1838x over previous
import functools

import jax
import jax.numpy as jnp
from jax.experimental import pallas as pl
from jax.experimental.pallas import tpu as pltpu

B, L, S, H, G, D = 2, 4096, 128, 4, 4, 128
TOPK = 16
BLOCK = 32
WINDOW = 32
SM_SCALE = 1.0 / (D ** 0.5)

TL = 512

SPLITS = ((0, 1, 16), (1, 2, 32), (2, 4, 64), (4, 8, 128))


def _topk_kernel(q_ref, k_ref, out_ref, idx_ref, *, lt_start, s_sub):
    lt = lt_start + pl.program_id(1)

    l_row = lt * TL + jax.lax.broadcasted_iota(jnp.int32, (1, TL), 1)
    lim = ((l_row + 1) // BLOCK) - 1
    s_iota = jax.lax.broadcasted_iota(jnp.int32, (s_sub, TL), 0)
    smask = s_iota >= lim
    neg_inf = jnp.float32(-jnp.inf)

    for h in range(H):
        kb = k_ref[0, h, :s_sub, :]
        scores_g = []
        for g in range(G):
            qg = q_ref[0, :, h, g, :]
            scores_g.append(jax.lax.dot_general(
                kb, qg, (((1,), (1,)), ((), ())),
                preferred_element_type=jnp.float32))

        pooled = jnp.maximum(jnp.maximum(scores_g[0], scores_g[1]),
                             jnp.maximum(scores_g[2], scores_g[3]))
        pooled = jnp.where(smask, neg_inf, pooled)

        idx_rows = []
        out_rows = [[] for _ in range(G)]
        for _ in range(TOPK):
            m = jnp.max(pooled, axis=0, keepdims=True)
            i = jnp.min(jnp.where(pooled == m, s_iota, s_sub), axis=0,
                        keepdims=True)
            onehot = s_iota == i
            invalid = jnp.isneginf(m)
            idx_rows.append(jnp.where(invalid, -1, i))
            for g in range(G):
                v = jnp.sum(jnp.where(onehot, scores_g[g], 0.0), axis=0,
                            keepdims=True)
                out_rows[g].append(jnp.where(invalid, 0.0, v * SM_SCALE))
            pooled = jnp.where(onehot, neg_inf, pooled)

        idx_ref[0, h, :, :] = jnp.concatenate(idx_rows, axis=0)
        for g in range(G):
            out_ref[0, h, g, :, :] = jnp.concatenate(out_rows[g], axis=0)


@jax.jit
def kernel(q, k):
    kp = jnp.transpose(k, (0, 2, 1, 3))
    out_parts, idx_parts = [], []
    for lt_start, lt_end, s_sub in SPLITS:
        nlt = lt_end - lt_start
        l_sub = nlt * TL
        out_t, idx_t = pl.pallas_call(
            functools.partial(_topk_kernel, lt_start=lt_start, s_sub=s_sub),
            grid=(B, nlt),
            in_specs=[
                pl.BlockSpec((1, TL, H, G, D),
                             lambda b, lt, s=lt_start: (b, s + lt, 0, 0, 0)),
                pl.BlockSpec((1, H, S, D), lambda b, lt: (b, 0, 0, 0)),
            ],
            out_specs=[
                pl.BlockSpec((1, H, G, TOPK, TL), lambda b, lt: (b, 0, 0, 0, lt)),
                pl.BlockSpec((1, H, TOPK, TL), lambda b, lt: (b, 0, 0, lt)),
            ],
            out_shape=[
                jax.ShapeDtypeStruct((B, H, G, TOPK, l_sub), jnp.float32),
                jax.ShapeDtypeStruct((B, H, TOPK, l_sub), jnp.int32),
            ],
            compiler_params=pltpu.CompilerParams(
                dimension_semantics=("parallel", "parallel")),
        )(q, kp)
        out_parts.append(out_t)
        idx_parts.append(idx_t)
    out_t = jnp.concatenate(out_parts, axis=-1)
    idx_t = jnp.concatenate(idx_parts, axis=-1)
    out_scores = jnp.transpose(out_t, (0, 4, 1, 2, 3))
    idx = jnp.transpose(idx_t, (0, 3, 1, 2))
    return out_scores, idx


# speedup vs baseline: 124.3560x; 1.1160x over previous
import functools

import jax
import jax.numpy as jnp
from jax.experimental import pallas as pl
from jax.experimental.pallas import tpu as pltpu

B, L, S, H, G, D = 2, 4096, 128, 4, 4, 128
TOPK = 16
BLOCK = 32
WINDOW = 32
SM_SCALE = 1.0 / (D ** 0.5)

TL = 256

SPLITS = ((0, 2, 16), (2, 4, 32), (4, 8, 64), (8, 16, 128))


def _topk_kernel(q_ref, k_ref, out_ref, idx_ref, *, lt_start, s_sub):
    lt = lt_start + pl.program_id(1)

    l_row = lt * TL + jax.lax.broadcasted_iota(jnp.int32, (1, TL), 1)
    lim = ((l_row + 1) // BLOCK) - 1
    s_iota = jax.lax.broadcasted_iota(jnp.int32, (s_sub, TL), 0)
    smask = s_iota >= lim
    neg_inf = jnp.float32(-jnp.inf)

    for h in range(H):
        kb = k_ref[0, h, :s_sub, :]
        scores_g = []
        for g in range(G):
            qg = q_ref[0, :, h, g, :]
            scores_g.append(jax.lax.dot_general(
                kb, qg, (((1,), (1,)), ((), ())),
                preferred_element_type=jnp.float32))

        pooled = jnp.maximum(jnp.maximum(scores_g[0], scores_g[1]),
                             jnp.maximum(scores_g[2], scores_g[3]))
        pooled = jnp.where(smask, neg_inf, pooled)

        idx_rows = []
        out_rows = [[] for _ in range(G)]
        for _ in range(TOPK):
            m = jnp.max(pooled, axis=0, keepdims=True)
            i = jnp.min(jnp.where(pooled == m, s_iota, s_sub), axis=0,
                        keepdims=True)
            onehot = s_iota == i
            invalid = jnp.isneginf(m)
            idx_rows.append(jnp.where(invalid, -1, i))
            for g in range(G):
                v = jnp.sum(jnp.where(onehot, scores_g[g], 0.0), axis=0,
                            keepdims=True)
                out_rows[g].append(jnp.where(invalid, 0.0, v * SM_SCALE))
            pooled = jnp.where(onehot, neg_inf, pooled)

        idx_ref[0, h, :, :] = jnp.concatenate(idx_rows, axis=0)
        for g in range(G):
            out_ref[0, h, g, :, :] = jnp.concatenate(out_rows[g], axis=0)


@jax.jit
def kernel(q, k):
    kp = jnp.transpose(k, (0, 2, 1, 3))
    out_parts, idx_parts = [], []
    for lt_start, lt_end, s_sub in SPLITS:
        nlt = lt_end - lt_start
        l_sub = nlt * TL
        out_t, idx_t = pl.pallas_call(
            functools.partial(_topk_kernel, lt_start=lt_start, s_sub=s_sub),
            grid=(B, nlt),
            in_specs=[
                pl.BlockSpec((1, TL, H, G, D),
                             lambda b, lt, s=lt_start: (b, s + lt, 0, 0, 0)),
                pl.BlockSpec((1, H, S, D), lambda b, lt: (b, 0, 0, 0)),
            ],
            out_specs=[
                pl.BlockSpec((1, H, G, TOPK, TL), lambda b, lt: (b, 0, 0, 0, lt)),
                pl.BlockSpec((1, H, TOPK, TL), lambda b, lt: (b, 0, 0, lt)),
            ],
            out_shape=[
                jax.ShapeDtypeStruct((B, H, G, TOPK, l_sub), jnp.float32),
                jax.ShapeDtypeStruct((B, H, TOPK, l_sub), jnp.int32),
            ],
            compiler_params=pltpu.CompilerParams(
                dimension_semantics=("parallel", "parallel")),
        )(q, kp)
        out_parts.append(out_t)
        idx_parts.append(idx_t)
    out_t = jnp.concatenate(out_parts, axis=-1)
    idx_t = jnp.concatenate(idx_parts, axis=-1)
    out_scores = jnp.transpose(out_t, (0, 4, 1, 2, 3))
    idx = jnp.transpose(idx_t, (0, 3, 1, 2))
    return out_scores, idx
